# Initial kernel scaffold; baseline (speedup 1.0000x reference)
import jax
import jax.numpy as jnp
from jax.experimental import pallas as pl


def kernel(Position, pos_embed_weight):
    raise NotImplementedError("write your pallas kernel here")


# speedup vs baseline: 1.1938x; 1.1938x over previous
import functools

import jax
import jax.numpy as jnp
from jax import lax
from jax.experimental import pallas as pl
from jax.experimental.pallas import tpu as pltpu
from jax.experimental.pallas import tpu_sc as plsc

_NUM_EMB = 130
_DIM = 2048
_B = 4096 * 50
_NC = 2
_NS = 16
_NW = _NC * _NS
_BPW = _B // _NW
_CH = 16
_NBUF = 2
_NCHUNK = _BPW // _CH
_NGROUP = _NCHUNK // _NBUF


def _embed_body(table_hbm, idx_hbm, out_hbm,
                idx_v, buf0, buf1, g0, g1, w0, w1):
    c = lax.axis_index("c")
    s = lax.axis_index("s")
    wid = s * _NC + c
    base = wid * _BPW

    pltpu.sync_copy(idx_hbm.at[pl.ds(base, _BPW)], idx_v)

    bufs = (buf0, buf1)
    gsem = (g0, g1)
    wsem = (w0, w1)

    def gather_start(j, b):
        pltpu.async_copy(table_hbm.at[idx_v.at[pl.ds(j * _CH, _CH)]],
                         bufs[b], gsem[b])

    def gather_wait(j, b):
        pltpu.make_async_copy(table_hbm.at[idx_v.at[pl.ds(j * _CH, _CH)]],
                              bufs[b], gsem[b]).wait()

    def write_start(j, b):
        pltpu.async_copy(bufs[b], out_hbm.at[pl.ds(base + j * _CH, _CH)],
                         wsem[b])

    def write_wait(j, b):
        pltpu.make_async_copy(bufs[b],
                              out_hbm.at[pl.ds(base + j * _CH, _CH)],
                              wsem[b]).wait()

    for b in range(_NBUF):
        gather_start(b, b)

    def group(g, carry):
        for b in range(_NBUF):
            j = g * _NBUF + b
            gather_wait(j, b)
            write_start(j, b)
            jn = j + _NBUF

            @pl.when(jn < _NCHUNK)
            def _():
                write_wait(j, b)
                gather_start(jn, b)
        return carry

    lax.fori_loop(0, _NGROUP, group, 0)

    for b in range(_NBUF):
        write_wait(_NCHUNK - _NBUF + b, b)


@functools.partial(
    pl.kernel,
    out_type=jax.ShapeDtypeStruct((_B, _DIM), jnp.float32),
    mesh=plsc.VectorSubcoreMesh(core_axis_name="c", subcore_axis_name="s"),
    scratch_types=[
        pltpu.VMEM((_BPW,), jnp.int32),
        pltpu.VMEM((_CH, _DIM), jnp.float32),
        pltpu.VMEM((_CH, _DIM), jnp.float32),
        pltpu.SemaphoreType.DMA,
        pltpu.SemaphoreType.DMA,
        pltpu.SemaphoreType.DMA,
        pltpu.SemaphoreType.DMA,
    ],
)
def _embed_lookup(table_hbm, idx_hbm, out_hbm, *scratch):
    _embed_body(table_hbm, idx_hbm, out_hbm, *scratch)


def kernel(Position, pos_embed_weight):
    idx = Position.reshape(-1).astype(jnp.int32)
    out = _embed_lookup(pos_embed_weight, idx)
    return out.reshape(Position.shape + (_DIM,))


# speedup vs baseline: 1.6192x; 1.3563x over previous
import functools

import jax
import jax.numpy as jnp
from jax import lax
from jax.experimental import pallas as pl
from jax.experimental.pallas import tpu as pltpu
from jax.experimental.pallas import tpu_sc as plsc

_NUM_EMB = 130
_DIM = 2048
_E0 = 4096
_E1 = 50
_B = _E0 * _E1
_NC = 2
_NS = 16
_NW = _NC * _NS
_BPW = _B // _NW
_SPW = _E0 // _NW


def _embed_body(table_hbm, idx_hbm, out_hbm, idx_v, buf, gsem, wsem):
    c = lax.axis_index("c")
    s = lax.axis_index("s")
    wid = s * _NC + c
    slab0 = wid * _SPW

    pltpu.sync_copy(idx_hbm.at[pl.ds(slab0, _SPW), :], idx_v)

    def gather_start(sl):
        pltpu.async_copy(table_hbm.at[idx_v.at[sl]], buf, gsem)

    def gather_wait(sl):
        pltpu.make_async_copy(table_hbm.at[idx_v.at[sl]], buf, gsem).wait()

    def write_start(sl):
        pltpu.async_copy(buf, out_hbm.at[slab0 + sl], wsem)

    def write_wait(sl):
        pltpu.make_async_copy(buf, out_hbm.at[slab0 + sl], wsem).wait()

    def slab_loop(sl, carry):
        gather_start(sl)
        gather_wait(sl)
        write_start(sl)
        write_wait(sl)
        return carry

    lax.fori_loop(0, _SPW, slab_loop, 0)


@functools.partial(
    pl.kernel,
    out_type=jax.ShapeDtypeStruct((_E0, _E1, _DIM), jnp.float32),
    mesh=plsc.VectorSubcoreMesh(core_axis_name="c", subcore_axis_name="s"),
    scratch_types=[
        pltpu.VMEM((_SPW, _E1), jnp.int32),
        pltpu.VMEM((_E1, _DIM), jnp.float32),
        pltpu.SemaphoreType.DMA,
        pltpu.SemaphoreType.DMA,
    ],
)
def _embed_lookup(table_hbm, idx_hbm, out_hbm, *scratch):
    _embed_body(table_hbm, idx_hbm, out_hbm, *scratch)


def kernel(Position, pos_embed_weight):
    idx = Position.astype(jnp.int32)
    return _embed_lookup(pos_embed_weight, idx)
